# baseline (speedup 1.0000x reference)
import functools

import jax
import jax.numpy as jnp
from jax import lax
from jax.experimental import pallas as pl
from jax.experimental.pallas import tpu as pltpu
from jax.experimental.pallas import tpu_sc as plsc

N = 10000
E = 320000
D_IN = 128
D_OUT = 128
NC = 2
NS = 16
NT = NC * NS
CPT = D_OUT // NT
KE = 512
NWIN = -(-E // KE)
NWIN_PAD = -(-NWIN // 2) * 2
E_PAD = NWIN_PAD * KE
NGRP = KE // 16


def _mm_body(x_ref, w_ref, o_ref):
    o_ref[...] = jnp.dot(x_ref[...], w_ref[...],
                         preferred_element_type=jnp.float32)


def _matmul(features, W):
    bm = 1000
    return pl.pallas_call(
        _mm_body,
        grid=(N // bm,),
        in_specs=[
            pl.BlockSpec((bm, D_IN), lambda i: (i, 0)),
            pl.BlockSpec((D_IN, D_OUT), lambda i: (0, 0)),
        ],
        out_specs=pl.BlockSpec((bm, D_OUT), lambda i: (i, 0)),
        out_shape=jax.ShapeDtypeStruct((N, D_OUT), jnp.float32),
    )(features, W)


def _core_ids():
    return lax.axis_index("c"), lax.axis_index("s")


def _sc_spmm(supportT, edata, adj3d):
    mesh = plsc.VectorSubcoreMesh(
        core_axis_name="c", subcore_axis_name="s", num_cores=NC, num_subcores=NS
    )

    @functools.partial(
        pl.kernel,
        out_type=jax.ShapeDtypeStruct((D_OUT, N), jnp.float32),
        mesh=mesh,
        compiler_params=pltpu.CompilerParams(
            use_tc_tiling_on_sc=False, needs_layout_passes=False),
        scratch_types=[
            pltpu.VMEM((CPT, N), jnp.float32),
            pltpu.VMEM((CPT, N), jnp.float32),
            pltpu.VMEM((2, KE), jnp.int32),
            pltpu.VMEM((2, KE), jnp.int32),
            pltpu.VMEM((KE,), jnp.float32),
            pltpu.VMEM((KE,), jnp.float32),
            pltpu.SemaphoreType.DMA,
            pltpu.SemaphoreType.DMA,
        ],
    )
    def spmm(supT_hbm, edata_hbm, adj_hbm, out_hbm,
             supt, acct, ebuf0, ebuf1, abuf0, abuf1, sem0, sem1):
        c, s = _core_ids()
        base = (c * NS + s) * CPT

        pltpu.sync_copy(supT_hbm.at[pl.ds(base, CPT)], supt)
        zero = jnp.zeros((16,), jnp.float32)

        def zcol(i, _):
            for j in range(CPT):
                acct[j, pl.ds(i * 16, 16)] = zero
            return 0

        lax.fori_loop(0, N // 16, zcol, 0)

        bufs = ((ebuf0, abuf0, sem0), (ebuf1, abuf1, sem1))

        def start_edata(w, b):
            eb, ab, se = bufs[b]
            pltpu.async_copy(edata_hbm.at[w], eb, se)
            pltpu.async_copy(adj_hbm.at[w, 0], ab, se)

        def wait_edata(w, b):
            eb, ab, se = bufs[b]
            pltpu.make_async_copy(edata_hbm.at[w], eb, se).wait()
            pltpu.make_async_copy(adj_hbm.at[w, 0], ab, se).wait()

        jvecs = [jnp.full((16,), j, jnp.int32) for j in range(CPT)]

        def process(w, b):
            eb, ab, _ = bufs[b]
            wait_edata(w, b)

            def grp(g, _):
                row_v = eb[0, pl.ds(g * 16, 16)]
                col_v = eb[1, pl.ds(g * 16, 16)]
                adj_v = ab[pl.ds(g * 16, 16)]
                for j in range(CPT):
                    vals = plsc.load_gather(supt, [jvecs[j], col_v]) * adj_v
                    plsc.addupdate_scatter(acct, [jvecs[j], row_v], vals)
                return 0

            lax.fori_loop(0, NGRP, grp, 0)

            @pl.when(w + 2 < NWIN_PAD)
            def _():
                start_edata(w + 2, b)

        start_edata(0, 0)
        start_edata(1, 1)

        def pair_body(p, _):
            process(2 * p, 0)
            process(2 * p + 1, 1)
            return 0

        lax.fori_loop(0, NWIN_PAD // 2, pair_body, 0)

        def rcol(i, _):
            for j in range(CPT):
                v = acct[j, pl.ds(i * 16, 16)]
                acct[j, pl.ds(i * 16, 16)] = jnp.maximum(v, 0.0)
            return 0

        lax.fori_loop(0, N // 16, rcol, 0)
        pltpu.sync_copy(acct, out_hbm.at[pl.ds(base, CPT)])

    return spmm(supportT, edata, adj3d)


def _pack_edges(edge_index, adj_values):
    pad = E_PAD - E
    row = edge_index[0]
    col = edge_index[1]
    if pad:
        spread = (jnp.arange(pad, dtype=jnp.int32) * 521) % N
        row = jnp.concatenate([row, spread])
        col = jnp.concatenate([col, spread])
        adj_values = jnp.concatenate(
            [adj_values, jnp.zeros((pad,), jnp.float32)]
        )
    packed = jnp.stack(
        [row.reshape(NWIN_PAD, KE), col.reshape(NWIN_PAD, KE)], axis=1
    )
    return packed, adj_values.reshape(NWIN_PAD, 1, KE)


def kernel(features, edge_index, adj_values, W):
    support = _matmul(features, W)
    edata, adj3d = _pack_edges(edge_index, adj_values)
    outT = _sc_spmm(support.T, edata, adj3d)
    return outT.T


# speedup vs baseline: 1.4846x; 1.4846x over previous
import functools

import jax
import jax.numpy as jnp
from jax import lax
from jax.experimental import pallas as pl
from jax.experimental.pallas import tpu as pltpu
from jax.experimental.pallas import tpu_sc as plsc

N = 10000
E = 320000
D_IN = 128
D_OUT = 128
NC = 2
NS = 16
NT = NC * NS
CPT = D_OUT // NT
KE = 512
NWIN = -(-E // KE)
NWIN_PAD = -(-NWIN // 2) * 2
E_PAD = NWIN_PAD * KE
NGRP = KE // 16


def _mm_body(x_ref, w_ref, o_ref):
    o_ref[...] = jnp.dot(x_ref[...], w_ref[...],
                         preferred_element_type=jnp.float32)


def _matmul(features, W):
    bm = 1000
    return pl.pallas_call(
        _mm_body,
        grid=(N // bm,),
        in_specs=[
            pl.BlockSpec((bm, D_IN), lambda i: (i, 0)),
            pl.BlockSpec((D_IN, D_OUT), lambda i: (0, 0)),
        ],
        out_specs=pl.BlockSpec((bm, D_OUT), lambda i: (i, 0)),
        out_shape=jax.ShapeDtypeStruct((N, D_OUT), jnp.float32),
    )(features, W)


def _core_ids():
    return lax.axis_index("c"), lax.axis_index("s")


def _sc_spmm(supportT, edata, adj3d):
    mesh = plsc.VectorSubcoreMesh(
        core_axis_name="c", subcore_axis_name="s", num_cores=NC, num_subcores=NS
    )

    @functools.partial(
        pl.kernel,
        out_type=jax.ShapeDtypeStruct((D_OUT, N), jnp.float32),
        mesh=mesh,
        compiler_params=pltpu.CompilerParams(
            use_tc_tiling_on_sc=False, needs_layout_passes=False),
        scratch_types=[
            pltpu.VMEM((CPT, N), jnp.float32),
            pltpu.VMEM((CPT, N), jnp.float32),
            pltpu.VMEM((2, KE), jnp.int32),
            pltpu.VMEM((2, KE), jnp.int32),
            pltpu.VMEM((KE,), jnp.float32),
            pltpu.VMEM((KE,), jnp.float32),
            pltpu.SemaphoreType.DMA,
            pltpu.SemaphoreType.DMA,
        ],
    )
    def spmm(supT_hbm, edata_hbm, adj_hbm, out_hbm,
             supt, acct, ebuf0, ebuf1, abuf0, abuf1, sem0, sem1):
        c, s = _core_ids()
        base = (c * NS + s) * CPT

        pltpu.sync_copy(supT_hbm.at[pl.ds(base, CPT)], supt)
        zero = jnp.zeros((16,), jnp.float32)

        def zcol(i, _):
            for j in range(CPT):
                acct[j, pl.ds(i * 16, 16)] = zero
            return 0

        lax.fori_loop(0, N // 16, zcol, 0)

        bufs = ((ebuf0, abuf0, sem0), (ebuf1, abuf1, sem1))

        def start_edata(w, b):
            eb, ab, se = bufs[b]
            pltpu.async_copy(edata_hbm.at[w], eb, se)
            pltpu.async_copy(adj_hbm.at[w, 0], ab, se)

        def wait_edata(w, b):
            eb, ab, se = bufs[b]
            pltpu.make_async_copy(edata_hbm.at[w], eb, se).wait()
            pltpu.make_async_copy(adj_hbm.at[w, 0], ab, se).wait()

        jvecs = [jnp.full((16,), j, jnp.int32) for j in range(CPT)]

        def process(w, b):
            eb, ab, _ = bufs[b]
            wait_edata(w, b)

            def grp(g, _):
                for h in range(2):
                    gg = g * 2 + h
                    row_v = eb[0, pl.ds(gg * 16, 16)]
                    col_v = eb[1, pl.ds(gg * 16, 16)]
                    adj_v = ab[pl.ds(gg * 16, 16)]
                    vals = [plsc.load_gather(supt, [jvecs[j], col_v])
                            for j in range(CPT)]
                    scaled = [v * adj_v for v in vals]
                    for j in range(CPT):
                        plsc.addupdate_scatter(acct, [jvecs[j], row_v],
                                               scaled[j])
                return 0

            lax.fori_loop(0, NGRP // 2, grp, 0)

            @pl.when(w + 2 < NWIN_PAD)
            def _():
                start_edata(w + 2, b)

        start_edata(0, 0)
        start_edata(1, 1)

        def pair_body(p, _):
            process(2 * p, 0)
            process(2 * p + 1, 1)
            return 0

        lax.fori_loop(0, NWIN_PAD // 2, pair_body, 0)

        def rcol(i, _):
            for j in range(CPT):
                v = acct[j, pl.ds(i * 16, 16)]
                acct[j, pl.ds(i * 16, 16)] = jnp.maximum(v, 0.0)
            return 0

        lax.fori_loop(0, N // 16, rcol, 0)
        pltpu.sync_copy(acct, out_hbm.at[pl.ds(base, CPT)])

    return spmm(supportT, edata, adj3d)


def _pack_edges(edge_index, adj_values):
    pad = E_PAD - E
    row = edge_index[0]
    col = edge_index[1]
    if pad:
        spread = (jnp.arange(pad, dtype=jnp.int32) * 521) % N
        row = jnp.concatenate([row, spread])
        col = jnp.concatenate([col, spread])
        adj_values = jnp.concatenate(
            [adj_values, jnp.zeros((pad,), jnp.float32)]
        )
    packed = jnp.stack(
        [row.reshape(NWIN_PAD, KE), col.reshape(NWIN_PAD, KE)], axis=1
    )
    return packed, adj_values.reshape(NWIN_PAD, 1, KE)


def kernel(features, edge_index, adj_values, W):
    support = _matmul(features, W)
    edata, adj3d = _pack_edges(edge_index, adj_values)
    outT = _sc_spmm(support.T, edata, adj3d)
    return outT.T
